# baseline (speedup 1.0000x reference)
import jax
import jax.numpy as jnp
from jax.experimental import pallas as pl
from jax.experimental.pallas import tpu as pltpu

_VMEM_LIMIT = 64 * 1024 * 1024


def _conv0_kernel(ze_ref, zo_ref, zte_ref, zto_ref, w_ref, b_ref, o_ref, ot_ref):
    acc = jnp.zeros((480, 128), jnp.float32)
    acct = jnp.zeros((60, 128), jnp.float32)
    for kh in range(12):
        m, p = kh // 2, kh % 2
        src = zo_ref if p else ze_ref
        srct = zto_ref if p else zte_ref
        acc = acc + jnp.dot(src[0, 8 * m: 8 * m + 480, :], w_ref[kh],
                            preferred_element_type=jnp.float32)
        acct = acct + jnp.dot(srct[0, m: m + 60, :], w_ref[kh],
                              preferred_element_type=jnp.float32)
    o_ref[0] = jnp.maximum(acc + b_ref[...], 0.0).astype(o_ref.dtype)
    ot_ref[0] = jnp.maximum(acct + b_ref[...], 0.0).astype(ot_ref.dtype)


def _conv1_kernel(z_ref, w_ref, b_ref, o_ref):
    acc = jnp.zeros((224, 32), jnp.float32)
    for kh in range(8):
        q, r = kh // 4, kh % 4
        a = z_ref[0, r, 16 * q: 16 * q + 224, :]
        acc = acc + jnp.dot(a, w_ref[kh], preferred_element_type=jnp.float32)
    o_ref[0] = jnp.maximum(acc + b_ref[...], 0.0).astype(o_ref.dtype)


def _conv23_kernel(z_ref, w2_ref, b2_ref, w3_ref, b3_ref, o_ref):
    acc2 = jnp.zeros((42, 64), jnp.float32)
    for kh in range(4):
        m, p = kh // 2, kh % 2
        a = z_ref[0, p, 7 * m: 7 * m + 42, :]
        acc2 = acc2 + jnp.dot(a, w2_ref[kh], preferred_element_type=jnp.float32)
    h2 = jnp.maximum(acc2 + b2_ref[...], 0.0).astype(jnp.bfloat16)
    h2p = jnp.concatenate([h2, jnp.zeros((14, 64), jnp.bfloat16)], axis=0)
    acc3 = jnp.zeros((28, 64), jnp.float32)
    for t in range(9):
        kh, kw = t // 3, t % 3
        a = h2p[7 * kh + kw: 7 * kh + kw + 28, :]
        acc3 = acc3 + jnp.dot(a, w3_ref[t], preferred_element_type=jnp.float32)
    o_ref[0] = jnp.maximum(acc3 + b3_ref[...], 0.0).astype(o_ref.dtype)


def _head_kernel(f_ref, w1_ref, b1_ref, w2_ref, b2_ref, o_ref):
    h = jnp.dot(f_ref[...], w1_ref[...], preferred_element_type=jnp.float32)
    h = jnp.maximum(h + b1_ref[...], 0.0).astype(jnp.bfloat16)
    o_ref[...] = jnp.dot(h, w2_ref[...], preferred_element_type=jnp.float32) + b2_ref[...]


def _batch_call(body, n_in, out_shapes, b):
    def mk_spec(arr, batched):
        shp = arr.shape
        if batched:
            blk = (1,) + shp[1:]
            zeros = (0,) * (len(shp) - 1)
            return pl.BlockSpec(blk, lambda i, _z=zeros: (i,) + _z)
        return pl.BlockSpec(shp, lambda i, _z=(0,) * len(shp): _z)

    def out_spec(shp):
        return pl.BlockSpec((1,) + shp, lambda i, _z=(0,) * len(shp): (i,) + _z)

    def call(*args):
        in_specs = [mk_spec(a, j < n_in) for j, a in enumerate(args)]
        return pl.pallas_call(
            body,
            out_shape=[jax.ShapeDtypeStruct((b,) + s, jnp.bfloat16)
                       for s in out_shapes],
            grid=(b,),
            in_specs=in_specs,
            out_specs=[out_spec(s) for s in out_shapes],
            compiler_params=pltpu.CompilerParams(
                dimension_semantics=("parallel",),
                vmem_limit_bytes=_VMEM_LIMIT,
            ),
        )(*args)
    return call


def kernel(w0, b0, w1, b1, w2, b2, w3, b3, w_l1, b_l1, w_l2, b_l2, x):
    B = x.shape[0]

    xb = x.astype(jnp.bfloat16)
    xp = jnp.pad(xb, ((0, 0), (0, 0), (0, 0), (0, 60)))
    zg = jnp.stack([xp[:, :, :, 64 * g: 64 * g + 80] for g in range(9)],
                   axis=3)
    zg = zg.transpose(0, 2, 3, 1, 4)
    zgm = zg[:, :, :8]
    ze = zgm[:, 0::2].reshape(B, 520, 240)
    zo = zgm[:, 1::2].reshape(B, 520, 240)
    zt = zg[:, :, 8]
    zte = zt[:, 0::2].reshape(B, 65, 240)
    zto = zt[:, 1::2].reshape(B, 65, 240)

    w0r = w0.reshape(3, 12, 12, 16)
    w0u = jnp.stack([jnp.pad(w0r, ((0, 0), (0, 0), (8 * u, 68 - 8 * u), (0, 0)))
                     for u in range(8)], axis=3)
    w0g = w0u.transpose(1, 0, 2, 3, 4).reshape(12, 240, 128)
    b0g = jnp.tile(b0, (1, 8))

    h0m, h0t = _batch_call(_conv0_kernel, 4, [(480, 128), (60, 128)], B)(
        ze, zo, zte, zto, w0g, b0g)
    h0 = jnp.concatenate([h0m.reshape(B, 60, 64, 16),
                          h0t.reshape(B, 60, 8, 16)[:, :, :4]], axis=2)

    h0b = h0.reshape(B, 60, 17, 64)
    z1 = jnp.concatenate([h0b[:, :, :16], h0b[:, :, 1:]], axis=-1)
    z1s = jnp.stack([z1[:, r::4] for r in range(4)], axis=1)
    z1s = z1s.reshape(B, 4, 240, 128)
    w1t = w1.reshape(16, 8, 8, 32).transpose(1, 2, 0, 3).reshape(8, 128, 32)

    h1, = _batch_call(_conv1_kernel, 1, [(224, 32)], B)(z1s, w1t, b1)
    h1 = h1.reshape(B, 14, 16, 32)

    h1b = h1.reshape(B, 14, 8, 64)
    z2 = jnp.concatenate([h1b[:, :, :7], h1b[:, :, 1:]], axis=-1)
    z2s = jnp.stack([z2[:, p::2] for p in range(2)], axis=1)
    z2s = z2s.reshape(B, 2, 49, 128)
    w2t = w2.reshape(32, 4, 4, 64).transpose(1, 2, 0, 3).reshape(4, 128, 64)
    w3t = w3.reshape(64, 3, 3, 64).transpose(1, 2, 0, 3).reshape(9, 64, 64)

    h3, = _batch_call(_conv23_kernel, 1, [(28, 64)], B)(z2s, w2t, b2, w3t, b3)

    feats = h3.reshape(B, 1792)
    w1f = jnp.pad(w_l1.reshape(4, 5, 64, 512),
                  ((0, 0), (0, 2), (0, 0), (0, 0))).reshape(1792, 512)

    out = pl.pallas_call(
        _head_kernel,
        out_shape=jax.ShapeDtypeStruct((B, 2), jnp.float32),
        grid=(1,),
        in_specs=[
            pl.BlockSpec((B, 1792), lambda i: (0, 0)),
            pl.BlockSpec((1792, 512), lambda i: (0, 0)),
            pl.BlockSpec((1, 512), lambda i: (0, 0)),
            pl.BlockSpec((512, 2), lambda i: (0, 0)),
            pl.BlockSpec((1, 2), lambda i: (0, 0)),
        ],
        out_specs=pl.BlockSpec((B, 2), lambda i: (0, 0)),
        compiler_params=pltpu.CompilerParams(
            dimension_semantics=("arbitrary",),
            vmem_limit_bytes=_VMEM_LIMIT,
        ),
    )(feats, w1f, b_l1, w_l2, b_l2)
    return out


# speedup vs baseline: 1.3485x; 1.3485x over previous
import jax
import jax.numpy as jnp
from jax.experimental import pallas as pl
from jax.experimental.pallas import tpu as pltpu

_VMEM_LIMIT = 64 * 1024 * 1024


def _convnet_kernel(ze_ref, zo_ref, zte_ref, zto_ref, w0_ref, b0_ref,
                    w1_ref, b1_ref, w2_ref, b2_ref, w3_ref, b3_ref, o_ref):
    acc = jnp.zeros((480, 128), jnp.float32)
    acct = jnp.zeros((60, 128), jnp.float32)
    for kh in range(12):
        m, p = kh // 2, kh % 2
        src = zo_ref if p else ze_ref
        srct = zto_ref if p else zte_ref
        acc = acc + jnp.dot(src[0, 8 * m: 8 * m + 480, :], w0_ref[kh],
                            preferred_element_type=jnp.float32)
        acct = acct + jnp.dot(srct[0, m: m + 60, :], w0_ref[kh],
                              preferred_element_type=jnp.float32)
    h0m = jnp.maximum(acc + b0_ref[...], 0.0).astype(jnp.bfloat16)
    h0t = jnp.maximum(acct + b0_ref[...], 0.0).astype(jnp.bfloat16)
    h0x = jnp.concatenate([h0m.reshape(60, 8, 128),
                           h0t.reshape(60, 1, 128)], axis=1)

    h0x4 = h0x.reshape(15, 4, 9, 128)
    acc1e = jnp.zeros((112, 32), jnp.float32)
    acc1o = jnp.zeros((112, 32), jnp.float32)
    for kh in range(8):
        q, r = kh // 4, kh % 4
        rows = h0x4[q: q + 14, r]
        a_e = rows[:, 0:8, :].reshape(112, 128)
        a_o = jnp.concatenate([rows[:, 0:8, 64:], rows[:, 1:9, :64]],
                              axis=-1).reshape(112, 128)
        acc1e = acc1e + jnp.dot(a_e, w1_ref[kh],
                                preferred_element_type=jnp.float32)
        acc1o = acc1o + jnp.dot(a_o, w1_ref[kh],
                                preferred_element_type=jnp.float32)
    h1e = jnp.maximum(acc1e + b1_ref[...], 0.0).astype(jnp.bfloat16)
    h1o = jnp.maximum(acc1o + b1_ref[...], 0.0).astype(jnp.bfloat16)

    E = h1e.reshape(14, 8, 32)
    O = h1o.reshape(14, 8, 32)
    z2 = jnp.concatenate([E[:, :7], O[:, :7], E[:, 1:8], O[:, 1:8]],
                         axis=-1)
    z2r = z2.reshape(7, 2, 7, 128)
    acc2 = jnp.zeros((42, 64), jnp.float32)
    for kh in range(4):
        m, p = kh // 2, kh % 2
        a = z2r[m: m + 6, p].reshape(42, 128)
        acc2 = acc2 + jnp.dot(a, w2_ref[kh], preferred_element_type=jnp.float32)
    h2 = jnp.maximum(acc2 + b2_ref[...], 0.0).astype(jnp.bfloat16)

    h2p = jnp.concatenate([h2, jnp.zeros((14, 64), jnp.bfloat16)], axis=0)
    acc3 = jnp.zeros((28, 64), jnp.float32)
    for t in range(9):
        kh, kw = t // 3, t % 3
        a = h2p[7 * kh + kw: 7 * kh + kw + 28, :]
        acc3 = acc3 + jnp.dot(a, w3_ref[t], preferred_element_type=jnp.float32)
    o_ref[0] = jnp.maximum(acc3 + b3_ref[...], 0.0).astype(o_ref.dtype)


def _head_kernel(f_ref, w1_ref, b1_ref, w2_ref, b2_ref, o_ref):
    h = jnp.dot(f_ref[...], w1_ref[...], preferred_element_type=jnp.float32)
    h = jnp.maximum(h + b1_ref[...], 0.0).astype(jnp.bfloat16)
    o_ref[...] = jnp.dot(h, w2_ref[...], preferred_element_type=jnp.float32) + b2_ref[...]


def _batch_call(body, n_in, out_shapes, b):
    def mk_spec(arr, batched):
        shp = arr.shape
        if batched:
            blk = (1,) + shp[1:]
            zeros = (0,) * (len(shp) - 1)
            return pl.BlockSpec(blk, lambda i, _z=zeros: (i,) + _z)
        return pl.BlockSpec(shp, lambda i, _z=(0,) * len(shp): _z)

    def out_spec(shp):
        return pl.BlockSpec((1,) + shp, lambda i, _z=(0,) * len(shp): (i,) + _z)

    def call(*args):
        in_specs = [mk_spec(a, j < n_in) for j, a in enumerate(args)]
        return pl.pallas_call(
            body,
            out_shape=[jax.ShapeDtypeStruct((b,) + s, jnp.bfloat16)
                       for s in out_shapes],
            grid=(b,),
            in_specs=in_specs,
            out_specs=[out_spec(s) for s in out_shapes],
            compiler_params=pltpu.CompilerParams(
                dimension_semantics=("parallel",),
                vmem_limit_bytes=_VMEM_LIMIT,
            ),
        )(*args)
    return call


def kernel(w0, b0, w1, b1, w2, b2, w3, b3, w_l1, b_l1, w_l2, b_l2, x):
    B = x.shape[0]

    xb = x.astype(jnp.bfloat16)
    xp = jnp.pad(xb, ((0, 0), (0, 0), (0, 0), (0, 60)))
    zg = jnp.stack([xp[:, :, :, 64 * g: 64 * g + 80] for g in range(9)],
                   axis=3)
    zg = zg.transpose(0, 2, 3, 1, 4)
    zgm = zg[:, :, :8]
    ze = zgm[:, 0::2].reshape(B, 520, 240)
    zo = zgm[:, 1::2].reshape(B, 520, 240)
    zt = zg[:, :, 8]
    zte = zt[:, 0::2].reshape(B, 65, 240)
    zto = zt[:, 1::2].reshape(B, 65, 240)

    w0r = w0.reshape(3, 12, 12, 16)
    w0u = jnp.stack([jnp.pad(w0r, ((0, 0), (0, 0), (8 * u, 68 - 8 * u), (0, 0)))
                     for u in range(8)], axis=3)
    w0g = w0u.transpose(1, 0, 2, 3, 4).reshape(12, 240, 128)
    b0g = jnp.tile(b0, (1, 8))

    w1t = w1.reshape(16, 8, 8, 32).transpose(1, 2, 0, 3).reshape(8, 128, 32)
    w2t = w2.reshape(32, 4, 4, 64).transpose(1, 2, 0, 3).reshape(4, 128, 64)
    w3t = w3.reshape(64, 3, 3, 64).transpose(1, 2, 0, 3).reshape(9, 64, 64)

    h3, = _batch_call(_convnet_kernel, 4, [(28, 64)], B)(
        ze, zo, zte, zto, w0g, b0g, w1t, b1, w2t, b2, w3t, b3)

    feats = h3.reshape(B, 1792)
    w1f = jnp.pad(w_l1.reshape(4, 5, 64, 512),
                  ((0, 0), (0, 2), (0, 0), (0, 0))).reshape(1792, 512)

    out = pl.pallas_call(
        _head_kernel,
        out_shape=jax.ShapeDtypeStruct((B, 2), jnp.float32),
        grid=(1,),
        in_specs=[
            pl.BlockSpec((B, 1792), lambda i: (0, 0)),
            pl.BlockSpec((1792, 512), lambda i: (0, 0)),
            pl.BlockSpec((1, 512), lambda i: (0, 0)),
            pl.BlockSpec((512, 2), lambda i: (0, 0)),
            pl.BlockSpec((1, 2), lambda i: (0, 0)),
        ],
        out_specs=pl.BlockSpec((B, 2), lambda i: (0, 0)),
        compiler_params=pltpu.CompilerParams(
            dimension_semantics=("arbitrary",),
            vmem_limit_bytes=_VMEM_LIMIT,
        ),
    )(feats, w1f, b_l1, w_l2, b_l2)
    return out
